# baseline (speedup 1.0000x reference)
import jax
import jax.numpy as jnp
from jax import lax
from jax.experimental import pallas as pl
from jax.experimental.pallas import tpu as pltpu
from jax.experimental.pallas import tpu_sc as plsc

N = 50000
E = 800000
HID = 64

NTILES = 16
HALF = N // 2
QUARTER = N // 4
RPT = 1600
ACC_ROWS = NTILES * RPT
RPT4 = 800
ACC4_ROWS = NTILES * RPT4
EPT = E // NTILES
CH = 5120
NCH = 10
CE_LAST = EPT - CH
LB_LAST = ((NCH - 1) * CH - CE_LAST) // 16
WIN = 128
NB = 5
CHP = CH + NB * WIN
NWC = CHP // WIN
ZR = 40

_MESH = plsc.VectorSubcoreMesh(core_axis_name="c", subcore_axis_name="s")


def _zero_rows(buf, nrows):
    zero16 = jnp.zeros((16,), jnp.float32)
    ncol = buf.shape[1]

    def body(r, _):
        for j in range(ncol // 16):
            buf[r, pl.ds(j * 16, 16)] = zero16
        return 0

    lax.fori_loop(0, nrows, body, 0)


def _deg_body(ei_h, deg_h, csrc_h, cdst_h, counts_h,
              acc, src_st, dst_st, qsrc, qdst, ones, sidx, zbuf, cnt_st,
              ssem):
    c = lax.axis_index("c")
    t = lax.axis_index("s")
    lo = c * HALF
    lane = lax.iota(jnp.int32, 16)

    one16 = jnp.full((16,), 1.0, jnp.float32)
    for r in range(WIN):
        ones[r, pl.ds(0, 16)] = one16
    _zero_rows(zbuf, ZR)
    for i in range(RPT // ZR):
        pltpu.sync_copy(zbuf, acc.at[pl.ds(t * RPT + i * ZR, ZR), :])
    plsc.subcore_barrier()

    def fire(b):
        pltpu.async_copy(ones, acc.at[sidx.at[b]], ssem.at[b], add=True)

    def drain(b):
        pltpu.make_async_copy(ones, acc.at[sidx.at[b]], ssem.at[b]).wait()

    def chunk_body(ch, _):
        ce = t * EPT + jnp.minimum(ch * CH, CE_LAST)
        lb = jnp.where(ch == NCH - 1, LB_LAST, 0)
        pltpu.sync_copy(ei_h.at[0, pl.ds(ce, CH)], src_st)
        pltpu.sync_copy(ei_h.at[1, pl.ds(ce, CH)], dst_st)

        for q in range(2):
            qlo = lo + q * QUARTER

            def scan_body(i, cntv):
                d = dst_st[pl.ds(i * 16, 16)]
                sv = src_st[pl.ds(i * 16, 16)]
                m = (d >= qlo) & (d < qlo + QUARTER)
                mi = m.astype(jnp.int32)
                pos = cntv + plsc.cumsum(mi) - 1
                plsc.store_scatter(qsrc, [pos], sv, mask=m)
                plsc.store_scatter(qdst, [pos >> 7, pos & 127], d - qlo,
                                   mask=m)
                return cntv + plsc.all_reduce_population_count(m)

            kv = lax.fori_loop(lb, CH // 16, scan_body,
                               jnp.zeros((16,), jnp.int32))
            k = kv[0]

            def pad_body(j, _):
                qsrc[pl.ds(k + j * 16, 16)] = lane * 97 + t * 64 + j * 16
                pp = k + j * 16 + lane
                plsc.store_scatter(qdst, [pp >> 7, pp & 127],
                                   QUARTER + lane + (j & 7) * 16)
                return 0

            lax.fori_loop(0, NB * WIN // 16, pad_body, 0)

            plsc.store_scatter(cnt_st, [lane * 0 + q * 32 + ch], kv,
                               mask=lane == 0)
            pltpu.sync_copy(qsrc, csrc_h.at[c, t, q, ch])
            pltpu.sync_copy(qdst, cdst_h.at[c, t, q, ch])

            nwin = (k + WIN - 1) // WIN
            npr = (nwin + 1) // 2

            def build(b, w):
                for j in range(WIN // 16):
                    v = qdst[w, pl.ds(j * 16, 16)]
                    sidx[b, pl.ds(j * 16, 16)] = \
                        jnp.where(v < QUARTER, v + q * QUARTER, v + QUARTER)

            @pl.when(npr > 0)
            def _():
                for b in range(2):
                    build(b, b)
                    fire(b)

            def pbody(p, _):
                for b in range(2):
                    drain(b)
                    build(b, 2 * p + b)
                    fire(b)
                return 0

            lax.fori_loop(1, npr, pbody, 0)

            @pl.when(npr > 0)
            def _():
                for b in range(2):
                    drain(b)
        return 0

    lax.fori_loop(0, NCH, chunk_body, 0)

    plsc.subcore_barrier()
    r0 = t * RPT

    @pl.when(t < NTILES - 1)
    def _():
        pltpu.sync_copy(acc.at[pl.ds(r0, RPT), :],
                        deg_h.at[pl.ds(lo + r0, RPT), :])

    @pl.when(t == NTILES - 1)
    def _():
        last = HALF - (NTILES - 1) * RPT
        pltpu.sync_copy(acc.at[pl.ds(r0, last), :],
                        deg_h.at[pl.ds(lo + r0, last), :])

    pltpu.sync_copy(cnt_st, counts_h.at[c, t])


def _scatter_body(csrc_h, cdst_h, counts_h, y_h, z_h,
                  acc, ssrc, sdst, rows, zbuf, cnt_st, gsem, ssem,
                  stsem):
    c = lax.axis_index("c")
    t = lax.axis_index("s")

    pltpu.sync_copy(counts_h.at[c, t], cnt_st.at[pl.ds(0, 64)])

    def stage(q, ch, s):
        pltpu.async_copy(csrc_h.at[c, t, q, ch], ssrc.at[s], stsem.at[s])
        pltpu.async_copy(cdst_h.at[c, t, q, ch], sdst.at[s], stsem.at[s])

    def wait_stage(s):
        pltpu.make_async_copy(csrc_h.at[c, t, 0, 0], ssrc.at[s],
                              stsem.at[s]).wait()
        pltpu.make_async_copy(cdst_h.at[c, t, 0, 0], sdst.at[s],
                              stsem.at[s]).wait()

    def fire_gather(s, b, w):
        pltpu.async_copy(y_h.at[ssrc.at[s, pl.ds(w * WIN, WIN)]], rows.at[b],
                         gsem.at[b])

    def wait_gather(b):
        pltpu.make_async_copy(y_h.at[pl.ds(0, WIN), :], rows.at[b],
                              gsem.at[b]).wait()

    def fire_scatter(s, b, w):
        pltpu.async_copy(rows.at[b], acc.at[sdst.at[s, w]], ssem.at[b],
                         add=True)

    def wait_scatter(s, b):
        pltpu.make_async_copy(rows.at[b], acc.at[sdst.at[s, 0]],
                              ssem.at[b]).wait()

    for q in range(2):
        zbase = c * HALF + q * QUARTER
        _zero_rows(zbuf, ZR)
        for i in range(RPT4 // ZR):
            pltpu.sync_copy(zbuf, acc.at[pl.ds(t * RPT4 + i * ZR, ZR), :])
        plsc.subcore_barrier()

        stage(q, 0, 0)

        def pair_body(p, _):
            for s in range(2):
                ch = 2 * p + s
                wait_stage(s)

                @pl.when(ch + 1 < NCH)
                def _():
                    stage(q, ch + 1, s ^ 1)

                kwin = cnt_st[pl.ds(q * 32 + ch, 16)]
                k = kwin[0]
                nwin = (k + WIN - 1) // WIN
                ngr = (nwin + NB - 1) // NB

                @pl.when(ngr > 0)
                def _():
                    for b in range(NB):
                        fire_gather(s, b, b)

                def gbody(g, _):
                    for b in range(NB):
                        w = NB * g + b
                        wait_gather(b)
                        fire_scatter(s, b, w)
                    for b in range(NB):
                        wait_scatter(s, b)

                        @pl.when(g < ngr - 1)
                        def _():
                            fire_gather(s, b, NB * (g + 1) + b)
                    return 0

                lax.fori_loop(0, ngr, gbody, 0)
            return 0

        lax.fori_loop(0, NCH // 2, pair_body, 0)

        plsc.subcore_barrier()
        r0 = t * RPT4

        @pl.when(t < NTILES - 1)
        def _():
            pltpu.sync_copy(acc.at[pl.ds(r0, RPT4), :],
                            z_h.at[pl.ds(zbase + r0, RPT4), :])

        @pl.when(t == NTILES - 1)
        def _():
            last = QUARTER - (NTILES - 1) * RPT4
            pltpu.sync_copy(acc.at[pl.ds(r0, last), :],
                            z_h.at[pl.ds(zbase + r0, last), :])


_SC_PARAMS = pltpu.CompilerParams(use_tc_tiling_on_sc=False,
                                  needs_layout_passes=False)

_deg_call = pl.kernel(
    _deg_body,
    out_type=[jax.ShapeDtypeStruct((N, 16), jnp.float32),
              jax.ShapeDtypeStruct((2, NTILES, 2, NCH, CHP), jnp.int32),
              jax.ShapeDtypeStruct((2, NTILES, 2, NCH, NWC, WIN), jnp.int32),
              jax.ShapeDtypeStruct((2, NTILES, 64), jnp.int32)],
    mesh=_MESH,
    compiler_params=_SC_PARAMS,
    scratch_types=[
        pltpu.VMEM_SHARED((ACC_ROWS, 16), jnp.float32),
        pltpu.VMEM((CH,), jnp.int32),
        pltpu.VMEM((CH,), jnp.int32),
        pltpu.VMEM((CHP,), jnp.int32),
        pltpu.VMEM((NWC, WIN), jnp.int32),
        pltpu.VMEM((WIN, 16), jnp.float32),
        pltpu.VMEM((2, WIN), jnp.int32),
        pltpu.VMEM((ZR, 16), jnp.float32),
        pltpu.VMEM((64,), jnp.int32),
        pltpu.SemaphoreType.DMA((2,)),
    ],
)

_scatter_call = pl.kernel(
    _scatter_body,
    out_type=jax.ShapeDtypeStruct((N, HID), jnp.float32),
    mesh=_MESH,
    compiler_params=_SC_PARAMS,
    scratch_types=[
        pltpu.VMEM_SHARED((ACC4_ROWS, HID), jnp.float32),
        pltpu.VMEM((2, CHP), jnp.int32),
        pltpu.VMEM((2, NWC, WIN), jnp.int32),
        pltpu.VMEM((NB, WIN, HID), jnp.float32),
        pltpu.VMEM((ZR, HID), jnp.float32),
        pltpu.VMEM((80,), jnp.int32),
        pltpu.SemaphoreType.DMA((NB,)),
        pltpu.SemaphoreType.DMA((NB,)),
        pltpu.SemaphoreType.DMA((2,)),
    ],
)


BN = 2000
GRID = N // BN
_P = None


def _k0_body(x_ref, mk_ref, dg_ref, w0_ref, b0_ref, wr_ref, br_ref,
             y_ref, base_ref, dis_ref):
    deg = dg_ref[:, 0:1] + 1.0
    dis = lax.rsqrt(deg)
    h0 = x_ref[...] * mk_ref[...]
    u0 = jnp.dot(h0, w0_ref[...], precision=_P)
    y0 = dis * u0
    res = jnp.dot(h0, wr_ref[...], precision=_P) + br_ref[...]
    y_ref[...] = y0
    base_ref[...] = res + b0_ref[...] + dis * y0
    dis_ref[...] = dis


def _kmid_body(z_ref, base_ref, dis_ref, w_ref, b_ref, y_ref, nbase_ref):
    dis = dis_ref[...]
    h = jnp.maximum(dis * z_ref[...] + base_ref[...], 0.0)
    u = jnp.dot(h, w_ref[...], precision=_P)
    y = dis * u
    y_ref[...] = y
    nbase_ref[...] = h + dis * y + b_ref[...]


def _k3_body(z_ref, base_ref, dis_ref, w_ref, b_ref, o_ref):
    dis = dis_ref[...]
    h = jnp.maximum(dis * z_ref[...] + base_ref[...], 0.0)
    o_ref[...] = jnp.dot(h, w_ref[...], precision=_P) + b_ref[...]


def _row_spec(w):
    return pl.BlockSpec((BN, w), lambda i: (i, 0))


def _full_spec(r, c):
    return pl.BlockSpec((r, c), lambda i: (0, 0))


_k0_call = pl.pallas_call(
    _k0_body,
    grid=(GRID,),
    in_specs=[_row_spec(4), _row_spec(1), _row_spec(16),
              _full_spec(4, HID), _full_spec(1, HID),
              _full_spec(4, HID), _full_spec(1, HID)],
    out_specs=[_row_spec(HID), _row_spec(HID), _row_spec(1)],
    out_shape=[jax.ShapeDtypeStruct((N, HID), jnp.float32),
               jax.ShapeDtypeStruct((N, HID), jnp.float32),
               jax.ShapeDtypeStruct((N, 1), jnp.float32)],
)

_kmid_call = pl.pallas_call(
    _kmid_body,
    grid=(GRID,),
    in_specs=[_row_spec(HID), _row_spec(HID), _row_spec(1),
              _full_spec(HID, HID), _full_spec(1, HID)],
    out_specs=[_row_spec(HID), _row_spec(HID)],
    out_shape=[jax.ShapeDtypeStruct((N, HID), jnp.float32),
               jax.ShapeDtypeStruct((N, HID), jnp.float32)],
)

_k3_call = pl.pallas_call(
    _k3_body,
    grid=(GRID,),
    in_specs=[_row_spec(HID), _row_spec(HID), _row_spec(1),
              _full_spec(HID, HID), _full_spec(1, HID)],
    out_specs=_row_spec(HID),
    out_shape=jax.ShapeDtypeStruct((N, HID), jnp.float32),
)


@jax.jit
def kernel(x, edge_index, hidden_mask, W0, b0, Wr0, br0, W1, b1, W2, b2,
           Wf, bf):
    maskf = hidden_mask.astype(jnp.float32)[:, None]

    deg16, csrc, cdst, counts = _deg_call(edge_index)
    y0, base0, dis = _k0_call(x, maskf, deg16, W0, b0[None, :], Wr0,
                              br0[None, :])
    z0 = _scatter_call(csrc, cdst, counts, y0)
    y1, base1 = _kmid_call(z0, base0, dis, W1, b1[None, :])
    z1 = _scatter_call(csrc, cdst, counts, y1)
    y2, base2 = _kmid_call(z1, base1, dis, W2, b2[None, :])
    z2 = _scatter_call(csrc, cdst, counts, y2)
    x_out = _k3_call(z2, base2, dis, Wf, bf[None, :])
    return (x_out, hidden_mask)


# speedup vs baseline: 16.9901x; 16.9901x over previous
import jax
import jax.numpy as jnp
from jax import lax
from jax.experimental import pallas as pl
from jax.experimental.pallas import tpu as pltpu
from jax.experimental.pallas import tpu_sc as plsc

N = 50000
E = 800000
HID = 64

NTILES = 16
HALF = N // 2
QUARTER = N // 4
RPT = 1600
ACC_ROWS = NTILES * RPT
RPT4 = 800
ACC4_ROWS = NTILES * RPT4
EPT = E // NTILES
CH = 5120
NCH = 10
CE_LAST = EPT - CH
LB_LAST = ((NCH - 1) * CH - CE_LAST) // 16
WIN = 128
NB = 6
CHP = CH + NB * WIN
NWC = CHP // WIN
ZR = 40

_MESH = plsc.VectorSubcoreMesh(core_axis_name="c", subcore_axis_name="s")


def _zero_rows(buf, nrows):
    zero16 = jnp.zeros((16,), jnp.float32)
    ncol = buf.shape[1]

    def body(r, _):
        for j in range(ncol // 16):
            buf[r, pl.ds(j * 16, 16)] = zero16
        return 0

    lax.fori_loop(0, nrows, body, 0)


def _deg_body(ei_h, deg_h, csrc_h, cdst_h, counts_h,
              acc, src_st, dst_st, qsrc, qdst, ones, sidx, zbuf, dgbuf,
              accbuf, cnt_st, ssem):
    c = lax.axis_index("c")
    t = lax.axis_index("s")
    lo = c * HALF
    lane = lax.iota(jnp.int32, 16)

    one16 = jnp.full((16,), 1.0, jnp.float32)
    for r in range(WIN):
        ones[r, pl.ds(0, 16)] = one16
    _zero_rows(zbuf, ZR)
    for i in range(RPT // ZR):
        pltpu.sync_copy(zbuf, acc.at[pl.ds(t * RPT + i * ZR, ZR), :])
    plsc.subcore_barrier()

    def fire(b):
        pltpu.async_copy(ones, acc.at[sidx.at[b]], ssem.at[b], add=True)

    def drain(b):
        pltpu.make_async_copy(ones, acc.at[sidx.at[b]], ssem.at[b]).wait()

    def chunk_body(ch, _):
        ce = t * EPT + jnp.minimum(ch * CH, CE_LAST)
        lb = jnp.where(ch == NCH - 1, LB_LAST, 0)
        pltpu.sync_copy(ei_h.at[0, pl.ds(ce, CH)], src_st)
        pltpu.sync_copy(ei_h.at[1, pl.ds(ce, CH)], dst_st)

        for q in range(2):
            qlo = lo + q * QUARTER

            def scan_body(i, cntv):
                d = dst_st[pl.ds(i * 16, 16)]
                sv = src_st[pl.ds(i * 16, 16)]
                m = (d >= qlo) & (d < qlo + QUARTER)
                mi = m.astype(jnp.int32)
                pos = cntv + plsc.cumsum(mi) - 1
                plsc.store_scatter(qsrc, [pos], sv, mask=m)
                plsc.store_scatter(qdst, [pos >> 7, pos & 127], d - qlo,
                                   mask=m)
                return cntv + plsc.all_reduce_population_count(m)

            kv = lax.fori_loop(lb, CH // 16, scan_body,
                               jnp.zeros((16,), jnp.int32))
            k = kv[0]

            def pad_body(j, _):
                qsrc[pl.ds(k + j * 16, 16)] = lane * 97 + t * 64 + j * 16
                pp = k + j * 16 + lane
                plsc.store_scatter(qdst, [pp >> 7, pp & 127],
                                   QUARTER + lane + (j & 7) * 16)
                return 0

            lax.fori_loop(0, NB * WIN // 16, pad_body, 0)

            plsc.store_scatter(cnt_st, [lane * 0 + q * 32 + ch], kv,
                               mask=lane == 0)
            pltpu.sync_copy(qsrc, csrc_h.at[c, t, q, ch])
            pltpu.sync_copy(qdst, cdst_h.at[c, t, q, ch])

            nwin = (k + WIN - 1) // WIN
            npr = (nwin + 1) // 2

            def build(b, w):
                for j in range(WIN // 16):
                    v = qdst[w, pl.ds(j * 16, 16)]
                    sidx[b, pl.ds(j * 16, 16)] = \
                        jnp.where(v < QUARTER, v + q * QUARTER, v + QUARTER)

            @pl.when(npr > 0)
            def _():
                for b in range(2):
                    build(b, b)
                    fire(b)

            def pbody(p, _):
                for b in range(2):
                    drain(b)
                    build(b, 2 * p + b)
                    fire(b)
                return 0

            lax.fori_loop(1, npr, pbody, 0)

            @pl.when(npr > 0)
            def _():
                for b in range(2):
                    drain(b)
        return 0

    lax.fori_loop(0, NCH, chunk_body, 0)

    plsc.subcore_barrier()

    r0 = t * RPT
    base_dg = (lo + r0) // 2

    def emit(nbatch):
        for bi in range(nbatch):
            pltpu.sync_copy(acc.at[pl.ds(r0 + bi * 200, 200), :], accbuf)

            def wbody(j, _):
                va = accbuf[2 * j, pl.ds(0, 16)]
                vb = accbuf[2 * j + 1, pl.ds(0, 16)]
                for u in range(4):
                    dgbuf[j, pl.ds(u * 16, 16)] = va
                    dgbuf[j, pl.ds(64 + u * 16, 16)] = vb
                return 0

            lax.fori_loop(0, 100, wbody, 0)
            pltpu.sync_copy(dgbuf,
                            deg_h.at[pl.ds(base_dg + bi * 100, 100), :])

    @pl.when(t < NTILES - 1)
    def _():
        emit(RPT // 2 // 100)

    @pl.when(t == NTILES - 1)
    def _():
        emit((HALF - (NTILES - 1) * RPT) // 2 // 100)

    pltpu.sync_copy(cnt_st, counts_h.at[c, t])


def _scatter_body(csrc_h, cdst_h, counts_h, y_h, z_h,
                  acc, ssrc, sdst, rows, zbuf, cnt_st, gsem, ssem,
                  stsem):
    c = lax.axis_index("c")
    t = lax.axis_index("s")

    pltpu.sync_copy(counts_h.at[c, t], cnt_st.at[pl.ds(0, 64)])

    def stage(q, ch, s):
        pltpu.async_copy(csrc_h.at[c, t, q, ch], ssrc.at[s], stsem.at[s])
        pltpu.async_copy(cdst_h.at[c, t, q, ch], sdst.at[s], stsem.at[s])

    def wait_stage(s):
        pltpu.make_async_copy(csrc_h.at[c, t, 0, 0], ssrc.at[s],
                              stsem.at[s]).wait()
        pltpu.make_async_copy(cdst_h.at[c, t, 0, 0], sdst.at[s],
                              stsem.at[s]).wait()

    def fire_gather(s, b, w):
        pltpu.async_copy(y_h.at[ssrc.at[s, pl.ds(w * WIN, WIN)]], rows.at[b],
                         gsem.at[b])

    def wait_gather(b):
        pltpu.make_async_copy(y_h.at[pl.ds(0, WIN), :], rows.at[b],
                              gsem.at[b]).wait()

    def fire_scatter(s, b, w):
        pltpu.async_copy(rows.at[b], acc.at[sdst.at[s, w]], ssem.at[b],
                         add=True)

    def wait_scatter(s, b):
        pltpu.make_async_copy(rows.at[b], acc.at[sdst.at[s, 0]],
                              ssem.at[b]).wait()

    for q in range(2):
        zbase = c * HALF + q * QUARTER
        _zero_rows(zbuf, ZR)
        for i in range(RPT4 // ZR):
            pltpu.sync_copy(zbuf, acc.at[pl.ds(t * RPT4 + i * ZR, ZR), :])
        plsc.subcore_barrier()

        stage(q, 0, 0)

        def pair_body(p, _):
            for s in range(2):
                ch = 2 * p + s
                wait_stage(s)

                @pl.when(ch + 1 < NCH)
                def _():
                    stage(q, ch + 1, s ^ 1)

                kwin = cnt_st[pl.ds(q * 32 + ch, 16)]
                k = kwin[0]
                nwin = (k + WIN - 1) // WIN
                ngr = (nwin + NB - 1) // NB

                @pl.when(ngr > 0)
                def _():
                    for b in range(NB):
                        fire_gather(s, b, b)

                def gbody(g, _):
                    for b in range(NB):
                        w = NB * g + b
                        wait_gather(b)
                        fire_scatter(s, b, w)
                    for b in range(NB):
                        wait_scatter(s, b)

                        @pl.when(g < ngr - 1)
                        def _():
                            fire_gather(s, b, NB * (g + 1) + b)
                    return 0

                lax.fori_loop(0, ngr, gbody, 0)
            return 0

        lax.fori_loop(0, NCH // 2, pair_body, 0)

        plsc.subcore_barrier()
        r0 = t * RPT4

        @pl.when(t < NTILES - 1)
        def _():
            pltpu.sync_copy(acc.at[pl.ds(r0, RPT4), :],
                            z_h.at[pl.ds(zbase + r0, RPT4), :])

        @pl.when(t == NTILES - 1)
        def _():
            last = QUARTER - (NTILES - 1) * RPT4
            pltpu.sync_copy(acc.at[pl.ds(r0, last), :],
                            z_h.at[pl.ds(zbase + r0, last), :])


_SC_PARAMS = pltpu.CompilerParams(use_tc_tiling_on_sc=False,
                                  needs_layout_passes=False)

_deg_call = pl.kernel(
    _deg_body,
    out_type=[jax.ShapeDtypeStruct((N // 2, 128), jnp.float32),
              jax.ShapeDtypeStruct((2, NTILES, 2, NCH, CHP), jnp.int32),
              jax.ShapeDtypeStruct((2, NTILES, 2, NCH, NWC, WIN), jnp.int32),
              jax.ShapeDtypeStruct((2, NTILES, 64), jnp.int32)],
    mesh=_MESH,
    compiler_params=_SC_PARAMS,
    scratch_types=[
        pltpu.VMEM_SHARED((ACC_ROWS, 16), jnp.float32),
        pltpu.VMEM((CH,), jnp.int32),
        pltpu.VMEM((CH,), jnp.int32),
        pltpu.VMEM((CHP,), jnp.int32),
        pltpu.VMEM((NWC, WIN), jnp.int32),
        pltpu.VMEM((WIN, 16), jnp.float32),
        pltpu.VMEM((2, WIN), jnp.int32),
        pltpu.VMEM((ZR, 16), jnp.float32),
        pltpu.VMEM((100, 128), jnp.float32),
        pltpu.VMEM((200, 16), jnp.float32),
        pltpu.VMEM((64,), jnp.int32),
        pltpu.SemaphoreType.DMA((2,)),
    ],
)

_scatter_call = pl.kernel(
    _scatter_body,
    out_type=jax.ShapeDtypeStruct((N, HID), jnp.float32),
    mesh=_MESH,
    compiler_params=_SC_PARAMS,
    scratch_types=[
        pltpu.VMEM_SHARED((ACC4_ROWS, HID), jnp.float32),
        pltpu.VMEM((2, CHP), jnp.int32),
        pltpu.VMEM((2, NWC, WIN), jnp.int32),
        pltpu.VMEM((NB, WIN, HID), jnp.float32),
        pltpu.VMEM((ZR, HID), jnp.float32),
        pltpu.VMEM((80,), jnp.int32),
        pltpu.SemaphoreType.DMA((NB,)),
        pltpu.SemaphoreType.DMA((NB,)),
        pltpu.SemaphoreType.DMA((2,)),
    ],
)


N2 = N // 2
BP = 1000
GRID = N2 // BP
H2 = 2 * HID


def _k0_body(x2_ref, m8_ref, dg_ref, w0_ref, b0_ref, wr_ref, br_ref,
             y_ref, base_ref, dis_ref):
    dis = lax.rsqrt(dg_ref[...] + 1.0)
    h0 = x2_ref[...] * m8_ref[...]
    u0 = jnp.dot(h0, w0_ref[...])
    y0 = dis * u0
    res = jnp.dot(h0, wr_ref[...]) + br_ref[...]
    y_ref[...] = y0
    base_ref[...] = res + b0_ref[...] + dis * y0
    dis_ref[...] = dis


def _kmid_body(z_ref, base_ref, dis_ref, w_ref, b_ref, y_ref, nbase_ref):
    dis = dis_ref[...]
    h = jnp.maximum(dis * z_ref[...] + base_ref[...], 0.0)
    u = jnp.dot(h, w_ref[...])
    y = dis * u
    y_ref[...] = y
    nbase_ref[...] = h + dis * y + b_ref[...]


def _k3_body(z_ref, base_ref, dis_ref, w_ref, b_ref, o_ref):
    dis = dis_ref[...]
    h = jnp.maximum(dis * z_ref[...] + base_ref[...], 0.0)
    o_ref[...] = jnp.dot(h, w_ref[...]) + b_ref[...]


def _row_spec(w):
    return pl.BlockSpec((BP, w), lambda i: (i, 0))


def _full_spec(r, c):
    return pl.BlockSpec((r, c), lambda i: (0, 0))


_k0_call = pl.pallas_call(
    _k0_body,
    grid=(GRID,),
    in_specs=[_row_spec(8), _row_spec(8), _row_spec(H2),
              _full_spec(8, H2), _full_spec(1, H2),
              _full_spec(8, H2), _full_spec(1, H2)],
    out_specs=[_row_spec(H2), _row_spec(H2), _row_spec(H2)],
    out_shape=[jax.ShapeDtypeStruct((N2, H2), jnp.float32),
               jax.ShapeDtypeStruct((N2, H2), jnp.float32),
               jax.ShapeDtypeStruct((N2, H2), jnp.float32)],
)

_kmid_call = pl.pallas_call(
    _kmid_body,
    grid=(GRID,),
    in_specs=[_row_spec(H2), _row_spec(H2), _row_spec(H2),
              _full_spec(H2, H2), _full_spec(1, H2)],
    out_specs=[_row_spec(H2), _row_spec(H2)],
    out_shape=[jax.ShapeDtypeStruct((N2, H2), jnp.float32),
               jax.ShapeDtypeStruct((N2, H2), jnp.float32)],
)

_k3_call = pl.pallas_call(
    _k3_body,
    grid=(GRID,),
    in_specs=[_row_spec(H2), _row_spec(H2), _row_spec(H2),
              _full_spec(H2, H2), _full_spec(1, H2)],
    out_specs=_row_spec(H2),
    out_shape=jax.ShapeDtypeStruct((N2, H2), jnp.float32),
)


def _pair_w(W):
    z = jnp.zeros_like(W)
    return jnp.concatenate(
        [jnp.concatenate([W, z], 1), jnp.concatenate([z, W], 1)], 0)


def _pair_b(b):
    return jnp.concatenate([b, b])[None, :]


@jax.jit
def kernel(x, edge_index, hidden_mask, W0, b0, Wr0, br0, W1, b1, W2, b2,
           Wf, bf):
    x2 = jnp.reshape(x, (N2, 8))
    mask8 = jnp.reshape(
        jnp.broadcast_to(hidden_mask.astype(jnp.float32)[:, None], (N, 4)),
        (N2, 8))

    dg128, csrc, cdst, counts = _deg_call(edge_index)

    y0, base0, dis = _k0_call(x2, mask8, dg128, _pair_w(W0), _pair_b(b0),
                              _pair_w(Wr0), _pair_b(br0))
    z0 = _scatter_call(csrc, cdst, counts, jnp.reshape(y0, (N, HID)))
    y1, base1 = _kmid_call(jnp.reshape(z0, (N2, H2)), base0, dis,
                           _pair_w(W1), _pair_b(b1))
    z1 = _scatter_call(csrc, cdst, counts, jnp.reshape(y1, (N, HID)))
    y2, base2 = _kmid_call(jnp.reshape(z1, (N2, H2)), base1, dis,
                           _pair_w(W2), _pair_b(b2))
    z2 = _scatter_call(csrc, cdst, counts, jnp.reshape(y2, (N, HID)))
    x_out2 = _k3_call(jnp.reshape(z2, (N2, H2)), base2, dis,
                      _pair_w(Wf), _pair_b(bf))
    return (jnp.reshape(x_out2, (N, HID)), hidden_mask)
